# baseline (speedup 1.0000x reference)
import jax
import jax.numpy as jnp
from jax.experimental import pallas as pl
from jax.experimental.pallas import tpu as pltpu
from jax.experimental.pallas import tpu_sc as plsc

_B, _D = 32768, 1024
_TB = 2048
_LANES = 128
_S = 2048
_NC, _NS = 2, 16
_NW = _NC * _NS
_RPW = _S // _NW
_KC = _D // 16


def _tc_body(x_ref, w_ref, b_ref, o_ref):
    logits = jnp.dot(x_ref[...], w_ref[...],
                     preferred_element_type=jnp.float32)
    lt = logits.T + b_ref[...]

    r0 = lt[0:1, :]
    r1 = lt[1:2, :]
    r2 = lt[2:3, :]
    e0 = (r0 >= r1) & (r0 >= r2)
    e1 = (~e0) & (r1 >= r2)

    def arg3(c0, c1, c2):
        a0 = (c0 >= c1) & (c0 >= c2)
        a1 = (~a0) & (c1 >= c2)
        return jnp.where(a0, 0, jnp.where(a1, 1, 2)).astype(jnp.int32)

    a_left = arg3(lt[3:4, :], lt[4:5, :], lt[5:6, :])
    a_sup = arg3(lt[6:7, :], lt[7:8, :], lt[8:9, :])
    a_right = arg3(lt[9:10, :], lt[10:11, :], lt[11:12, :])

    final_arg = jnp.where(e0, a_left, jnp.where(e1, a_sup, a_right))
    relabeled = jnp.where(e0, 1, jnp.where(e1, 0, 2)).astype(jnp.int32)
    o_ref[...] = (relabeled * 3 + final_arg + 1).reshape(1, 1, _TB)


def _sc_body(x_hbm, wt_hbm, bvec_hbm, out_hbm, x_v, wt_v, bvec_v, out_v):
    c = jax.lax.axis_index("c")
    s = jax.lax.axis_index("s")
    wid = s * _NC + c
    base = wid * _RPW

    pltpu.sync_copy(wt_hbm, wt_v)
    pltpu.sync_copy(bvec_hbm, bvec_v)
    pltpu.sync_copy(x_hbm.at[pl.ds(base * _D, _RPW * _D)], x_v)

    iota = jax.lax.iota(jnp.int32, 16)
    idx_a = jnp.minimum(iota * 3, 12)
    idx_b = jnp.minimum(iota * 3 + 1, 13)
    idx_c = jnp.minimum(iota * 3 + 2, 14)
    zeros16 = iota * 0

    def lane_mask(d):
        return 1 - jnp.minimum(d * d, 1)

    def ge(x, y):
        return (jnp.sign(x - y).astype(jnp.int32) + 2) >> 1

    def hsum(a):
        for sh in (8, 4, 2, 1):
            a = a + a.at[(iota + sh) & 15].get(mode="promise_in_bounds")
        return a


    def route_row(v):
        va = v.at[idx_a].get(mode="promise_in_bounds")
        vb = v.at[idx_b].get(mode="promise_in_bounds")
        vc = v.at[idx_c].get(mode="promise_in_bounds")
        a0 = ge(va, vb) * ge(va, vc)
        argv = (1 - a0) * (2 - ge(vb, vc))
        e_v = argv.at[zeros16].get(mode="promise_in_bounds")
        f_v = argv.at[e_v + 1].get(mode="promise_in_bounds")
        rel = 1 - e_v + 3 * (e_v >> 1)
        return rel * 3 + f_v + 1

    def group(g, carry_outer):
        def row_pair(ii, res_acc):
            r0 = g * 16 + 2 * ii
            o0 = r0 * _D
            o1 = o0 + _D
            acc0 = [jnp.zeros((16,), jnp.float32) for _ in range(12)]
            acc1 = [jnp.zeros((16,), jnp.float32) for _ in range(12)]
            for k in range(_KC):
                xv0 = x_v[pl.ds(o0 + 16 * k, 16)]
                xv1 = x_v[pl.ds(o1 + 16 * k, 16)]
                for j in range(12):
                    wv = wt_v[pl.ds(j * _D + 16 * k, 16)]
                    acc0[j] = acc0[j] + xv0 * wv
                    acc1[j] = acc1[j] + xv1 * wv
            bvec = bvec_v[...]
            v0 = bvec
            v1 = bvec
            for j in range(12):
                lane_j = lane_mask(iota - j).astype(jnp.float32)
                v0 = v0 + lane_j * hsum(acc0[j])
                v1 = v1 + lane_j * hsum(acc1[j])
            m0 = lane_mask(iota - 2 * ii)
            m1 = lane_mask(iota - 2 * ii - 1)
            res_acc = res_acc + m0 * (route_row(v0) - res_acc)
            res_acc = res_acc + m1 * (route_row(v1) - res_acc)
            return res_acc

        res16 = jax.lax.fori_loop(0, 8, row_pair, iota * 0)
        out_v[pl.ds(g * 16, 16)] = res16
        return carry_outer

    jax.lax.fori_loop(0, _RPW // 16, group, 0)

    pltpu.sync_copy(out_v, out_hbm.at[pl.ds(base, _RPW)])


def kernel(xb, W_raw, b_raw, W_left, b_left, W_sup, b_sup, W_right, b_right):
    xb = xb.astype(jnp.float32)
    W = jnp.concatenate([W_raw, W_left, W_sup, W_right], axis=1)
    Wp = jnp.pad(W, ((0, 0), (0, _LANES - 12)))
    b = jnp.concatenate([b_raw, b_left, b_sup, b_right])
    bp = jnp.pad(b, (0, _LANES - 12)).reshape(_LANES, 1)

    def _round_bf16(x):
        bits = jax.lax.bitcast_convert_type(x, jnp.int32)
        t = bits + 0x7FFF + ((bits >> 16) & 1)
        return jax.lax.bitcast_convert_type(t & ~0xFFFF, jnp.float32)

    wt = _round_bf16(W.T)
    bvec = jnp.pad(b, (0, 4))
    xs = _round_bf16(xb[:_S]).reshape(_S * _D)

    sc_out = pl.kernel(
        _sc_body,
        out_type=jax.ShapeDtypeStruct((_S,), jnp.int32),
        mesh=plsc.VectorSubcoreMesh(core_axis_name="c", subcore_axis_name="s"),
        scratch_types=[
            pltpu.VMEM((_RPW * _D,), jnp.float32),
            pltpu.VMEM((12 * _D,), jnp.float32),
            pltpu.VMEM((16,), jnp.float32),
            pltpu.VMEM((_RPW,), jnp.int32),
        ],
    )(xs, wt.reshape(12 * _D), bvec)

    nb = (_B - _S) // _TB
    tc_out = pl.pallas_call(
        _tc_body,
        grid=(nb,),
        in_specs=[
            pl.BlockSpec((_TB, _D), lambda i: (i + _S // _TB, 0)),
            pl.BlockSpec((_D, _LANES), lambda i: (0, 0)),
            pl.BlockSpec((_LANES, 1), lambda i: (0, 0)),
        ],
        out_specs=pl.BlockSpec((1, 1, _TB), lambda i: (i, 0, 0)),
        out_shape=jax.ShapeDtypeStruct((nb, 1, _TB), jnp.int32),
    )(xb, Wp, bp)

    return jnp.concatenate([sc_out, tc_out.reshape((_B - _S,))])


# speedup vs baseline: 1.0056x; 1.0056x over previous
import jax
import jax.numpy as jnp
from jax.experimental import pallas as pl
from jax.experimental.pallas import tpu as pltpu
from jax.experimental.pallas import tpu_sc as plsc

_B, _D = 32768, 1024
_TB = 2048
_LANES = 128
_S = 2048
_NC, _NS = 2, 16
_NW = _NC * _NS
_RPW = _S // _NW
_KC = _D // 16


def _tc_body(x_ref, w_ref, b_ref, o_ref):
    logits = jnp.dot(x_ref[...], w_ref[...],
                     preferred_element_type=jnp.float32)
    lt = logits.T + b_ref[...]

    r0 = lt[0:1, :]
    r1 = lt[1:2, :]
    r2 = lt[2:3, :]
    e0 = (r0 >= r1) & (r0 >= r2)
    e1 = (~e0) & (r1 >= r2)

    def arg3(c0, c1, c2):
        a0 = (c0 >= c1) & (c0 >= c2)
        a1 = (~a0) & (c1 >= c2)
        return jnp.where(a0, 0, jnp.where(a1, 1, 2)).astype(jnp.int32)

    a_left = arg3(lt[3:4, :], lt[4:5, :], lt[5:6, :])
    a_sup = arg3(lt[6:7, :], lt[7:8, :], lt[8:9, :])
    a_right = arg3(lt[9:10, :], lt[10:11, :], lt[11:12, :])

    final_arg = jnp.where(e0, a_left, jnp.where(e1, a_sup, a_right))
    relabeled = jnp.where(e0, 1, jnp.where(e1, 0, 2)).astype(jnp.int32)
    o_ref[...] = (relabeled * 3 + final_arg + 1).reshape(1, 1, _TB)


def _sc_body(x_hbm, wt_hbm, bvec_hbm, out_hbm, x_v, wt_v, bvec_v, out_v):
    c = jax.lax.axis_index("c")
    s = jax.lax.axis_index("s")
    wid = s * _NC + c
    base = wid * _RPW

    pltpu.sync_copy(wt_hbm, wt_v)
    pltpu.sync_copy(bvec_hbm, bvec_v)
    pltpu.sync_copy(x_hbm.at[pl.ds(base * _D, _RPW * _D)], x_v)

    iota = jax.lax.iota(jnp.int32, 16)
    idx_a = jnp.minimum(iota * 3, 12)
    idx_b = jnp.minimum(iota * 3 + 1, 13)
    idx_c = jnp.minimum(iota * 3 + 2, 14)
    zeros16 = iota * 0

    def lane_mask(d):
        return 1 - jnp.minimum(d * d, 1)

    def ge(x, y):
        return (jnp.sign(x - y).astype(jnp.int32) + 2) >> 1

    def hsum(a):
        for sh in (8, 4, 2, 1):
            a = a + a.at[(iota + sh) & 15].get(mode="promise_in_bounds")
        return a


    def route_row(v):
        va = v.at[idx_a].get(mode="promise_in_bounds")
        vb = v.at[idx_b].get(mode="promise_in_bounds")
        vc = v.at[idx_c].get(mode="promise_in_bounds")
        a0 = ge(va, vb) * ge(va, vc)
        argv = (1 - a0) * (2 - ge(vb, vc))
        e_v = argv.at[zeros16].get(mode="promise_in_bounds")
        f_v = argv.at[e_v + 1].get(mode="promise_in_bounds")
        rel = 1 - e_v + 3 * (e_v >> 1)
        return rel * 3 + f_v + 1

    def group(g, carry_outer):
        def row_pair(ii, res_acc):
            r0 = g * 16 + 2 * ii
            o0 = r0 * _D
            o1 = o0 + _D
            acc0 = [jnp.zeros((16,), jnp.float32) for _ in range(12)]
            acc1 = [jnp.zeros((16,), jnp.float32) for _ in range(12)]
            for k in range(_KC):
                xv0 = x_v[pl.ds(o0 + 16 * k, 16)]
                xv1 = x_v[pl.ds(o1 + 16 * k, 16)]
                for j in range(12):
                    wv = wt_v[pl.ds(j * _D + 16 * k, 16)]
                    acc0[j] = acc0[j] + xv0 * wv
                    acc1[j] = acc1[j] + xv1 * wv
            bvec = bvec_v[...]
            v0 = bvec
            v1 = bvec
            for j in range(12):
                lane_j = lane_mask(iota - j).astype(jnp.float32)
                v0 = v0 + lane_j * hsum(acc0[j])
                v1 = v1 + lane_j * hsum(acc1[j])
            m0 = lane_mask(iota - 2 * ii)
            m1 = lane_mask(iota - 2 * ii - 1)
            res_acc = res_acc + m0 * (route_row(v0) - res_acc)
            res_acc = res_acc + m1 * (route_row(v1) - res_acc)
            return res_acc

        res16 = jax.lax.fori_loop(0, 8, row_pair, iota * 0)
        out_v[pl.ds(g * 16, 16)] = res16
        return carry_outer

    jax.lax.fori_loop(0, _RPW // 16, group, 0)

    pltpu.sync_copy(out_v, out_hbm.at[pl.ds(base, _RPW)])


def kernel(xb, W_raw, b_raw, W_left, b_left, W_sup, b_sup, W_right, b_right):
    xb = xb.astype(jnp.float32)
    W = jnp.concatenate([W_raw, W_left, W_sup, W_right], axis=1)
    Wp = jnp.pad(W, ((0, 0), (0, _LANES - 12)))
    b = jnp.concatenate([b_raw, b_left, b_sup, b_right])
    bp = jnp.pad(b, (0, _LANES - 12)).reshape(_LANES, 1)

    def _round_bf16(x):
        bits = jax.lax.bitcast_convert_type(x, jnp.int32)
        t = bits + 0x7FFF + ((bits >> 16) & 1)
        return jax.lax.bitcast_convert_type(t & ~0xFFFF, jnp.float32)

    wt = _round_bf16(W.T)
    bvec = jnp.pad(b, (0, 4))
    xs = _round_bf16(xb[:_S]).reshape(_S * _D)

    nb = (_B - _S) // _TB
    tc_out = pl.pallas_call(
        _tc_body,
        grid=(nb,),
        in_specs=[
            pl.BlockSpec((_TB, _D), lambda i: (i + _S // _TB, 0)),
            pl.BlockSpec((_D, _LANES), lambda i: (0, 0)),
            pl.BlockSpec((_LANES, 1), lambda i: (0, 0)),
        ],
        out_specs=pl.BlockSpec((1, 1, _TB), lambda i: (i, 0, 0)),
        out_shape=jax.ShapeDtypeStruct((nb, 1, _TB), jnp.int32),
    )(xb, Wp, bp)

    sc_out = pl.kernel(
        _sc_body,
        out_type=jax.ShapeDtypeStruct((_S,), jnp.int32),
        mesh=plsc.VectorSubcoreMesh(core_axis_name="c", subcore_axis_name="s"),
        scratch_types=[
            pltpu.VMEM((_RPW * _D,), jnp.float32),
            pltpu.VMEM((12 * _D,), jnp.float32),
            pltpu.VMEM((16,), jnp.float32),
            pltpu.VMEM((_RPW,), jnp.int32),
        ],
    )(xs, wt.reshape(12 * _D), bvec)

    return jnp.concatenate([sc_out, tc_out.reshape((_B - _S,))])


# speedup vs baseline: 3.2657x; 3.2474x over previous
import jax
import jax.numpy as jnp
from jax.experimental import pallas as pl

_B, _D = 32768, 1024
_TB = 2048
_LANES = 128


def _fused_body(x_ref, w_ref, b_ref, o_ref):
    logits = jnp.dot(x_ref[...], w_ref[...],
                     preferred_element_type=jnp.float32)
    lt = logits.T + b_ref[...]

    r0 = lt[0:1, :]
    r1 = lt[1:2, :]
    r2 = lt[2:3, :]
    e0 = (r0 >= r1) & (r0 >= r2)
    e1 = (~e0) & (r1 >= r2)

    def arg3(c0, c1, c2):
        a0 = (c0 >= c1) & (c0 >= c2)
        a1 = (~a0) & (c1 >= c2)
        return jnp.where(a0, 0, jnp.where(a1, 1, 2)).astype(jnp.int32)

    a_left = arg3(lt[3:4, :], lt[4:5, :], lt[5:6, :])
    a_sup = arg3(lt[6:7, :], lt[7:8, :], lt[8:9, :])
    a_right = arg3(lt[9:10, :], lt[10:11, :], lt[11:12, :])

    final_arg = jnp.where(e0, a_left, jnp.where(e1, a_sup, a_right))
    relabeled = jnp.where(e0, 1, jnp.where(e1, 0, 2)).astype(jnp.int32)
    o_ref[...] = (relabeled * 3 + final_arg + 1).reshape(1, 1, _TB)


def kernel(xb, W_raw, b_raw, W_left, b_left, W_sup, b_sup, W_right, b_right):
    xb = xb.astype(jnp.float32)
    W = jnp.concatenate([W_raw, W_left, W_sup, W_right], axis=1)
    Wp = jnp.pad(W, ((0, 0), (0, _LANES - 12)))
    b = jnp.concatenate([b_raw, b_left, b_sup, b_right])
    bp = jnp.pad(b, (0, _LANES - 12)).reshape(_LANES, 1)

    nb = _B // _TB
    out = pl.pallas_call(
        _fused_body,
        grid=(nb,),
        in_specs=[
            pl.BlockSpec((_TB, _D), lambda i: (i, 0)),
            pl.BlockSpec((_D, _LANES), lambda i: (0, 0)),
            pl.BlockSpec((_LANES, 1), lambda i: (0, 0)),
        ],
        out_specs=pl.BlockSpec((1, 1, _TB), lambda i: (i, 0, 0)),
        out_shape=jax.ShapeDtypeStruct((nb, 1, _TB), jnp.int32),
    )(xb, Wp, bp)
    return out.reshape(_B)
